# Initial kernel scaffold; baseline (speedup 1.0000x reference)
import jax
import jax.numpy as jnp
from jax.experimental import pallas as pl


def kernel(x, ei, ew, W1, b1, W2, b2):
    raise NotImplementedError("write your pallas kernel here")


# speedup vs baseline: 18.3868x; 18.3868x over previous
import functools

import jax
import jax.numpy as jnp
from jax import lax
from jax.experimental import pallas as pl
from jax.experimental.pallas import tpu as pltpu
from jax.experimental.pallas import tpu_sc as plsc

N = 10000
E = 320000
D = 64
CH = 80
ER = E // CH
NC = 2
NS = 16
NW = NC * NS
EPW = E // NW
RPW = EPW // CH
WR = 5
NWIN = RPW // WR
NPAD = 10240
NPT = N // NS
ZR = 125

_mesh = plsc.VectorSubcoreMesh(core_axis_name="c", subcore_axis_name="s")
_sc_params = pltpu.CompilerParams(use_tc_tiling_on_sc=False,
                                  needs_layout_passes=False)


def _deg_body(dst_hbm, ew_hbm, deg_out, dstb, ewb, zero_v, deg_sh, sem):
    cid = lax.axis_index("c")
    sid = lax.axis_index("s")
    wid = cid * NS + sid

    def _zfill(i, _):
        zero_v[pl.ds(i * 16, 16)] = jnp.zeros((16,), jnp.float32)
        return 0
    lax.fori_loop(0, 40, _zfill, 0)
    pltpu.sync_copy(zero_v, deg_sh.at[pl.ds(sid * 640, 640)])
    plsc.subcore_barrier()

    pltpu.sync_copy(dst_hbm.at[pl.ds(wid * RPW, RPW)], dstb)
    pltpu.sync_copy(ew_hbm.at[pl.ds(wid * RPW, RPW)], ewb)

    def _chunk(i, _):
        descs = []
        for r in range(WR):
            descs.append(pltpu.async_copy(
                ewb.at[i * WR + r], deg_sh.at[dstb.at[i * WR + r]], sem,
                add=True))
        for d in descs:
            d.wait()
        return 0
    lax.fori_loop(0, RPW // WR, _chunk, 0)

    plsc.subcore_barrier()
    pltpu.sync_copy(deg_sh.at[pl.ds(sid * 640, 640)],
                    deg_out.at[cid, pl.ds(sid * 640, 640)])


_deg_kernel = functools.partial(
    pl.kernel, _deg_body,
    out_type=jax.ShapeDtypeStruct((NC, NPAD), jnp.float32),
    mesh=_mesh,
    compiler_params=_sc_params,
    scratch_types=[
        pltpu.VMEM((RPW, CH), jnp.int32),
        pltpu.VMEM((RPW, CH), jnp.float32),
        pltpu.VMEM((640,), jnp.float32),
        pltpu.VMEM_SHARED((NPAD,), jnp.float32),
        pltpu.SemaphoreType.DMA,
    ],
)()


def _agg_body(compute_norm, *refs):
    if compute_norm:
        (h_hbm, src_hbm, dst_hbm, ew_hbm, dis_hbm, agg_out, norm_out,
         h_sh, agg_sh, srcb, dstb, wb, rows, zero_v, dis_v, normb,
         gsem, ssem) = refs
    else:
        (h_hbm, src_hbm, dst_hbm, norm_hbm, agg_out,
         h_sh, agg_sh, srcb, dstb, wb, rows, zero_v,
         gsem, ssem) = refs
    cid = lax.axis_index("c")
    sid = lax.axis_index("s")
    wid = cid * NS + sid

    pltpu.sync_copy(h_hbm.at[pl.ds(sid * NPT, NPT)],
                    h_sh.at[pl.ds(sid * NPT, NPT)])

    def _zfill(i, _):
        for c in range(D // 16):
            zero_v[i, pl.ds(c * 16, 16)] = jnp.zeros((16,), jnp.float32)
        return 0
    lax.fori_loop(0, ZR, _zfill, 0)
    for k in range(NPT // ZR):
        pltpu.sync_copy(zero_v, agg_sh.at[pl.ds(sid * NPT + k * ZR, ZR)])

    if compute_norm:
        pltpu.sync_copy(dis_hbm, dis_v)
    plsc.subcore_barrier()

    def _window(w, _):
        base = wid * RPW + w * WR
        pltpu.sync_copy(src_hbm.at[pl.ds(base, WR)], srcb)
        pltpu.sync_copy(dst_hbm.at[pl.ds(base, WR)], dstb)
        if compute_norm:
            pltpu.sync_copy(ew_hbm.at[pl.ds(base, WR)], wb)
            for r in range(WR):
                for c in range(CH // 16):
                    sl = pl.ds(c * 16, 16)
                    nv = (plsc.load_gather(dis_v, [srcb[r, sl]])
                          * wb[r, sl]
                          * plsc.load_gather(dis_v, [dstb[r, sl]]))
                    normb[r, sl] = nv
            pltpu.sync_copy(normb, norm_out.at[pl.ds(base, WR)])
            nref = normb
        else:
            pltpu.sync_copy(norm_hbm.at[pl.ds(base, WR)], wb)
            nref = wb

        gds = [pltpu.async_copy(h_sh.at[srcb.at[r]], rows.at[r], gsem)
               for r in range(WR)]
        sds = []
        for r in range(WR):
            gds[r].wait()

            def _scale(g, _):
                nv16 = nref[r, pl.ds(g * 16, 16)]
                for jj in range(16):
                    nvec = jnp.full((16,), nv16[jj], jnp.float32)
                    for c in range(D // 16):
                        sl = pl.ds(c * 16, 16)
                        j = g * 16 + jj
                        rows[r, j, sl] = rows[r, j, sl] * nvec
                return 0
            lax.fori_loop(0, CH // 16, _scale, 0)
            sds.append(pltpu.async_copy(
                rows.at[r], agg_sh.at[dstb.at[r]], ssem, add=True))
        for d in sds:
            d.wait()
        return 0
    lax.fori_loop(0, NWIN, _window, 0)

    plsc.subcore_barrier()
    for k in range(NPT // ZR):
        sl = pl.ds(sid * NPT + k * ZR, ZR)
        pltpu.sync_copy(agg_sh.at[sl], agg_out.at[cid, sl])


def _make_agg(compute_norm):
    out_type = [jax.ShapeDtypeStruct((NC, N, D), jnp.float32)]
    scratch = [
        pltpu.VMEM_SHARED((N, D), jnp.float32),
        pltpu.VMEM_SHARED((N, D), jnp.float32),
        pltpu.VMEM((WR, CH), jnp.int32),
        pltpu.VMEM((WR, CH), jnp.int32),
        pltpu.VMEM((WR, CH), jnp.float32),
        pltpu.VMEM((WR, CH, D), jnp.float32),
        pltpu.VMEM((ZR, D), jnp.float32),
    ]
    if compute_norm:
        out_type.append(jax.ShapeDtypeStruct((ER, CH), jnp.float32))
        scratch.append(pltpu.VMEM((N,), jnp.float32))
        scratch.append(pltpu.VMEM((WR, CH), jnp.float32))
    scratch.append(pltpu.SemaphoreType.DMA)
    scratch.append(pltpu.SemaphoreType.DMA)
    return functools.partial(
        pl.kernel, functools.partial(_agg_body, compute_norm),
        out_type=tuple(out_type), mesh=_mesh, scratch_types=scratch,
        compiler_params=_sc_params)()


_agg1 = _make_agg(True)
_agg2 = _make_agg(False)


def _prep_body(p0, p1, x, w1, dis_o, selfc_o, h1_o):
    deg = p0[...] + p1[...] + 1.0
    dis_o[...] = lax.rsqrt(deg)
    selfc_o[...] = 1.0 / deg
    h1_o[...] = jnp.dot(x[...], w1[...], preferred_element_type=jnp.float32)


def _mid_body(aggp, h, selfc, b, w2, h1_o, h2_o):
    pre = aggp[0] + aggp[1] + h[...] * selfc[...] + b[...]
    h1 = jnp.maximum(pre, 0.0)
    h1_o[...] = h1
    h2_o[...] = jnp.dot(h1, w2[...], preferred_element_type=jnp.float32)


def _final_body(aggp, h, selfc, b, h1, out_o):
    out_o[...] = aggp[0] + aggp[1] + h[...] * selfc[...] + b[...] + h1[...]


_prep = pl.pallas_call(
    _prep_body,
    out_shape=(
        jax.ShapeDtypeStruct((N, 1), jnp.float32),
        jax.ShapeDtypeStruct((N, 1), jnp.float32),
        jax.ShapeDtypeStruct((N, D), jnp.float32),
    ),
)

_mid = pl.pallas_call(
    _mid_body,
    out_shape=(
        jax.ShapeDtypeStruct((N, D), jnp.float32),
        jax.ShapeDtypeStruct((N, D), jnp.float32),
    ),
)

_final = pl.pallas_call(
    _final_body,
    out_shape=jax.ShapeDtypeStruct((N, D), jnp.float32),
)


def kernel(x, ei, ew, W1, b1, W2, b2):
    src = ei[0].astype(jnp.int32).reshape(ER, CH)
    dst = ei[1].astype(jnp.int32).reshape(ER, CH)
    ew2 = ew.reshape(ER, CH)

    deg_p = _deg_kernel(dst, ew2)
    p0 = deg_p[0, :N].reshape(N, 1)
    p1 = deg_p[1, :N].reshape(N, 1)
    dis, selfc, h1p = _prep(p0, p1, x, W1)

    agg1, norm = _agg1(h1p, src, dst, ew2, dis.reshape(N))
    h1, h2p = _mid(agg1, h1p, selfc, b1, W2)

    (agg2,) = _agg2(h2p, src, dst, norm)
    return _final(agg2, h2p, selfc, b2, h1)


# speedup vs baseline: 20.6697x; 1.1242x over previous
import functools

import jax
import jax.numpy as jnp
from jax import lax
from jax.experimental import pallas as pl
from jax.experimental.pallas import tpu as pltpu
from jax.experimental.pallas import tpu_sc as plsc

N = 10000
E = 320000
D = 64
CH = 80
ER = E // CH
NC = 2
NS = 16
NW = NC * NS
EPW = E // NW
RPW = EPW // CH
WR = 5
NWIN = RPW // WR
NPAD = 10240
NPT = N // NS
ZR = 125

_mesh = plsc.VectorSubcoreMesh(core_axis_name="c", subcore_axis_name="s")
_sc_params = pltpu.CompilerParams(use_tc_tiling_on_sc=False,
                                  needs_layout_passes=False)


def _deg_body(dst_hbm, ew_hbm, deg_out, dstb, ewb, zero_v, deg_sh, sem):
    cid = lax.axis_index("c")
    sid = lax.axis_index("s")
    wid = cid * NS + sid

    def _zfill(i, _):
        zero_v[pl.ds(i * 16, 16)] = jnp.zeros((16,), jnp.float32)
        return 0
    lax.fori_loop(0, 40, _zfill, 0)
    pltpu.sync_copy(zero_v, deg_sh.at[pl.ds(sid * 640, 640)])
    plsc.subcore_barrier()

    pltpu.sync_copy(dst_hbm.at[pl.ds(wid * RPW, RPW)], dstb)
    pltpu.sync_copy(ew_hbm.at[pl.ds(wid * RPW, RPW)], ewb)

    def _chunk(i, _):
        descs = []
        for r in range(WR):
            descs.append(pltpu.async_copy(
                ewb.at[i * WR + r], deg_sh.at[dstb.at[i * WR + r]], sem,
                add=True))
        for d in descs:
            d.wait()
        return 0
    lax.fori_loop(0, RPW // WR, _chunk, 0)

    plsc.subcore_barrier()
    pltpu.sync_copy(deg_sh.at[pl.ds(sid * 640, 640)],
                    deg_out.at[cid, pl.ds(sid * 640, 640)])


_deg_kernel = functools.partial(
    pl.kernel, _deg_body,
    out_type=jax.ShapeDtypeStruct((NC, NPAD), jnp.float32),
    mesh=_mesh,
    compiler_params=_sc_params,
    scratch_types=[
        pltpu.VMEM((RPW, CH), jnp.int32),
        pltpu.VMEM((RPW, CH), jnp.float32),
        pltpu.VMEM((640,), jnp.float32),
        pltpu.VMEM_SHARED((NPAD,), jnp.float32),
        pltpu.SemaphoreType.DMA,
    ],
)()


def _agg_body(compute_norm, *refs):
    if compute_norm:
        (h_hbm, src_hbm, dst_hbm, ew_hbm, dis_hbm, agg_out, norm_out,
         agg_sh, srcb, dstb, wb, rows, zero_v, dis_v,
         gsem, ssem) = refs
    else:
        (h_hbm, src_hbm, dst_hbm, norm_hbm, agg_out,
         agg_sh, srcb, dstb, wb, rows, zero_v,
         gsem, ssem) = refs
    cid = lax.axis_index("c")
    sid = lax.axis_index("s")
    wid = cid * NS + sid

    def _zfill(i, _):
        for c in range(D // 16):
            zero_v[i, pl.ds(c * 16, 16)] = jnp.zeros((16,), jnp.float32)
        return 0
    lax.fori_loop(0, ZR, _zfill, 0)
    for k in range(NPT // ZR):
        pltpu.sync_copy(zero_v, agg_sh.at[pl.ds(sid * NPT + k * ZR, ZR)])

    pltpu.sync_copy(src_hbm.at[pl.ds(wid * RPW, RPW)], srcb)
    pltpu.sync_copy(dst_hbm.at[pl.ds(wid * RPW, RPW)], dstb)
    if compute_norm:
        pltpu.sync_copy(ew_hbm.at[pl.ds(wid * RPW, RPW)], wb)
        pltpu.sync_copy(dis_hbm, dis_v)

        def _normf(r, _):
            for c in range(CH // 16):
                sl = pl.ds(c * 16, 16)
                wb[r, sl] = (plsc.load_gather(dis_v, [srcb[r, sl]])
                             * wb[r, sl]
                             * plsc.load_gather(dis_v, [dstb[r, sl]]))
            return 0
        lax.fori_loop(0, RPW, _normf, 0)
        pltpu.sync_copy(wb, norm_out.at[pl.ds(wid * RPW, RPW)])
    else:
        pltpu.sync_copy(norm_hbm.at[pl.ds(wid * RPW, RPW)], wb)
    plsc.subcore_barrier()

    def _window(w, _):
        base = w * WR
        gds = [pltpu.async_copy(h_hbm.at[srcb.at[base + r]], rows.at[r],
                                gsem)
               for r in range(WR)]
        sds = []
        for r in range(WR):
            gds[r].wait()

            def _scale(g, _):
                nv16 = wb[base + r, pl.ds(g * 16, 16)]
                for jj in range(16):
                    nvec = jnp.full((16,), nv16[jj], jnp.float32)
                    for c in range(D // 16):
                        sl = pl.ds(c * 16, 16)
                        j = g * 16 + jj
                        rows[r, j, sl] = rows[r, j, sl] * nvec
                return 0
            lax.fori_loop(0, CH // 16, _scale, 0)
            sds.append(pltpu.async_copy(
                rows.at[r], agg_sh.at[dstb.at[base + r]], ssem, add=True))
        for d in sds:
            d.wait()
        return 0
    lax.fori_loop(0, NWIN, _window, 0)

    plsc.subcore_barrier()
    for k in range(NPT // ZR):
        sl = pl.ds(sid * NPT + k * ZR, ZR)
        pltpu.sync_copy(agg_sh.at[sl], agg_out.at[cid, sl])


def _make_agg(compute_norm):
    out_type = [jax.ShapeDtypeStruct((NC, N, D), jnp.float32)]
    scratch = [
        pltpu.VMEM_SHARED((N, D), jnp.float32),
        pltpu.VMEM((RPW, CH), jnp.int32),
        pltpu.VMEM((RPW, CH), jnp.int32),
        pltpu.VMEM((RPW, CH), jnp.float32),
        pltpu.VMEM((WR, CH, D), jnp.float32),
        pltpu.VMEM((ZR, D), jnp.float32),
    ]
    if compute_norm:
        out_type.append(jax.ShapeDtypeStruct((ER, CH), jnp.float32))
        scratch.append(pltpu.VMEM((N,), jnp.float32))
    scratch.append(pltpu.SemaphoreType.DMA)
    scratch.append(pltpu.SemaphoreType.DMA)
    return functools.partial(
        pl.kernel, functools.partial(_agg_body, compute_norm),
        out_type=tuple(out_type), mesh=_mesh, scratch_types=scratch,
        compiler_params=_sc_params)()


_agg1 = _make_agg(True)
_agg2 = _make_agg(False)


def _prep_body(p0, p1, x, w1, dis_o, selfc_o, h1_o):
    deg = p0[...] + p1[...] + 1.0
    dis_o[...] = lax.rsqrt(deg)
    selfc_o[...] = 1.0 / deg
    h1_o[...] = jnp.dot(x[...], w1[...], preferred_element_type=jnp.float32)


def _mid_body(aggp, h, selfc, b, w2, h1_o, h2_o):
    pre = aggp[0] + aggp[1] + h[...] * selfc[...] + b[...]
    h1 = jnp.maximum(pre, 0.0)
    h1_o[...] = h1
    h2_o[...] = jnp.dot(h1, w2[...], preferred_element_type=jnp.float32)


def _final_body(aggp, h, selfc, b, h1, out_o):
    out_o[...] = aggp[0] + aggp[1] + h[...] * selfc[...] + b[...] + h1[...]


_prep = pl.pallas_call(
    _prep_body,
    out_shape=(
        jax.ShapeDtypeStruct((N, 1), jnp.float32),
        jax.ShapeDtypeStruct((N, 1), jnp.float32),
        jax.ShapeDtypeStruct((N, D), jnp.float32),
    ),
)

_mid = pl.pallas_call(
    _mid_body,
    out_shape=(
        jax.ShapeDtypeStruct((N, D), jnp.float32),
        jax.ShapeDtypeStruct((N, D), jnp.float32),
    ),
)

_final = pl.pallas_call(
    _final_body,
    out_shape=jax.ShapeDtypeStruct((N, D), jnp.float32),
)


def kernel(x, ei, ew, W1, b1, W2, b2):
    src = ei[0].astype(jnp.int32).reshape(ER, CH)
    dst = ei[1].astype(jnp.int32).reshape(ER, CH)
    ew2 = ew.reshape(ER, CH)

    deg_p = _deg_kernel(dst, ew2)
    p0 = deg_p[0, :N].reshape(N, 1)
    p1 = deg_p[1, :N].reshape(N, 1)
    dis, selfc, h1p = _prep(p0, p1, x, W1)

    agg1, norm = _agg1(h1p, src, dst, ew2, dis.reshape(N))
    h1, h2p = _mid(agg1, h1p, selfc, b1, W2)

    (agg2,) = _agg2(h2p, src, dst, norm)
    return _final(agg2, h2p, selfc, b2, h1)


# speedup vs baseline: 36.0104x; 1.7422x over previous
import functools

import jax
import jax.numpy as jnp
from jax import lax
from jax.experimental import pallas as pl
from jax.experimental.pallas import tpu as pltpu
from jax.experimental.pallas import tpu_sc as plsc

N = 10000
E = 320000
D = 64
CH = 80
ER = E // CH
NC = 2
NS = 16
NW = NC * NS
EPW = E // NW
RPW = EPW // CH
WR = 5
NWIN = RPW // WR
NPAD = 10240
NPT = N // NS
ZR = 125

_mesh = plsc.VectorSubcoreMesh(core_axis_name="c", subcore_axis_name="s")
_sc_params = pltpu.CompilerParams(use_tc_tiling_on_sc=False,
                                  needs_layout_passes=False)


def _deg_body(dst_hbm, ew_hbm, deg_out, dstb, ewb, zero_v, deg_sh, sem):
    cid = lax.axis_index("c")
    sid = lax.axis_index("s")
    wid = cid * NS + sid

    def _zfill(i, _):
        zero_v[pl.ds(i * 16, 16)] = jnp.zeros((16,), jnp.float32)
        return 0
    lax.fori_loop(0, 40, _zfill, 0)
    pltpu.sync_copy(zero_v, deg_sh.at[pl.ds(sid * 640, 640)])
    plsc.subcore_barrier()

    pltpu.sync_copy(dst_hbm.at[pl.ds(wid * RPW, RPW)], dstb)
    pltpu.sync_copy(ew_hbm.at[pl.ds(wid * RPW, RPW)], ewb)

    def _chunk(i, _):
        descs = []
        for r in range(WR):
            descs.append(pltpu.async_copy(
                ewb.at[i * WR + r], deg_sh.at[dstb.at[i * WR + r]], sem,
                add=True))
        for d in descs:
            d.wait()
        return 0
    lax.fori_loop(0, RPW // WR, _chunk, 0)

    plsc.subcore_barrier()
    pltpu.sync_copy(deg_sh.at[pl.ds(sid * 640, 640)],
                    deg_out.at[cid, pl.ds(sid * 640, 640)])


_deg_kernel = functools.partial(
    pl.kernel, _deg_body,
    out_type=jax.ShapeDtypeStruct((NC, NPAD), jnp.float32),
    mesh=_mesh,
    compiler_params=_sc_params,
    scratch_types=[
        pltpu.VMEM((RPW, CH), jnp.int32),
        pltpu.VMEM((RPW, CH), jnp.float32),
        pltpu.VMEM((640,), jnp.float32),
        pltpu.VMEM_SHARED((NPAD,), jnp.float32),
        pltpu.SemaphoreType.DMA,
    ],
)()


def _agg_body(hs_hbm, src_hbm, dst_hbm, ew_hbm, agg_out,
              agg_sh, srcb, dstb, wb, rows, zero_v, gsem, ssem):
    cid = lax.axis_index("c")
    sid = lax.axis_index("s")
    wid = cid * NS + sid

    def _zfill(i, _):
        for c in range(D // 16):
            zero_v[i, pl.ds(c * 16, 16)] = jnp.zeros((16,), jnp.float32)
        return 0
    lax.fori_loop(0, ZR, _zfill, 0)
    for k in range(NPT // ZR):
        pltpu.sync_copy(zero_v, agg_sh.at[pl.ds(sid * NPT + k * ZR, ZR)])

    pltpu.sync_copy(src_hbm.at[pl.ds(wid * RPW, RPW)], srcb)
    pltpu.sync_copy(dst_hbm.at[pl.ds(wid * RPW, RPW)], dstb)
    pltpu.sync_copy(ew_hbm.at[pl.ds(wid * RPW, RPW)], wb)
    plsc.subcore_barrier()

    def _window(w, _):
        base = w * WR
        gds = [pltpu.async_copy(hs_hbm.at[srcb.at[base + r]], rows.at[r],
                                gsem)
               for r in range(WR)]
        sds = []
        for r in range(WR):
            gds[r].wait()

            @plsc.parallel_loop(0, CH // 16)
            def _scale(g):
                nv16 = wb[base + r, pl.ds(g * 16, 16)]
                for jj in range(16):
                    nvec = jnp.full((16,), nv16[jj], jnp.float32)
                    for c in range(D // 16):
                        sl = pl.ds(c * 16, 16)
                        j = g * 16 + jj
                        rows[r, j, sl] = rows[r, j, sl] * nvec

            sds.append(pltpu.async_copy(
                rows.at[r], agg_sh.at[dstb.at[base + r]], ssem, add=True))
        for d in sds:
            d.wait()
        return 0
    lax.fori_loop(0, NWIN, _window, 0)

    plsc.subcore_barrier()
    for k in range(NPT // ZR):
        sl = pl.ds(sid * NPT + k * ZR, ZR)
        pltpu.sync_copy(agg_sh.at[sl], agg_out.at[cid, sl])


_agg = functools.partial(
    pl.kernel, _agg_body,
    out_type=jax.ShapeDtypeStruct((NC, N, D), jnp.float32),
    mesh=_mesh,
    compiler_params=_sc_params,
    scratch_types=[
        pltpu.VMEM_SHARED((N, D), jnp.float32),
        pltpu.VMEM((RPW, CH), jnp.int32),
        pltpu.VMEM((RPW, CH), jnp.int32),
        pltpu.VMEM((RPW, CH), jnp.float32),
        pltpu.VMEM((WR, CH, D), jnp.float32),
        pltpu.VMEM((ZR, D), jnp.float32),
        pltpu.SemaphoreType.DMA,
        pltpu.SemaphoreType.DMA,
    ],
)()


def _prep_body(p0, p1, x, w1, dis_o, hs1_o):
    deg = p0[...] + p1[...] + 1.0
    dis = lax.rsqrt(deg)
    dis_o[...] = dis
    hs1_o[...] = dis * jnp.dot(x[...], w1[...],
                               preferred_element_type=jnp.float32)


def _mid_body(aggp, hs1, dis, b, w2, h1_o, hs2_o):
    h1 = jnp.maximum((aggp[0] + aggp[1] + hs1[...]) * dis[...] + b[...], 0.0)
    h1_o[...] = h1
    hs2_o[...] = dis[...] * jnp.dot(h1, w2[...],
                                    preferred_element_type=jnp.float32)


def _final_body(aggp, hs2, dis, b, h1, out_o):
    out_o[...] = ((aggp[0] + aggp[1] + hs2[...]) * dis[...] + b[...]
                  + h1[...])


_prep = pl.pallas_call(
    _prep_body,
    out_shape=(
        jax.ShapeDtypeStruct((N, 1), jnp.float32),
        jax.ShapeDtypeStruct((N, D), jnp.float32),
    ),
)

_mid = pl.pallas_call(
    _mid_body,
    out_shape=(
        jax.ShapeDtypeStruct((N, D), jnp.float32),
        jax.ShapeDtypeStruct((N, D), jnp.float32),
    ),
)

_final = pl.pallas_call(
    _final_body,
    out_shape=jax.ShapeDtypeStruct((N, D), jnp.float32),
)


def kernel(x, ei, ew, W1, b1, W2, b2):
    src = ei[0].astype(jnp.int32).reshape(ER, CH)
    dst = ei[1].astype(jnp.int32).reshape(ER, CH)
    ew2 = ew.reshape(ER, CH)

    deg_p = _deg_kernel(dst, ew2)
    p0 = deg_p[0, :N].reshape(N, 1)
    p1 = deg_p[1, :N].reshape(N, 1)
    dis, hs1 = _prep(p0, p1, x, W1)

    agg1 = _agg(hs1, src, dst, ew2)
    h1, hs2 = _mid(agg1, hs1, dis, b1, W2)

    agg2 = _agg(hs2, src, dst, ew2)
    return _final(agg2, hs2, dis, b2, h1)
